# Initial kernel scaffold; baseline (speedup 1.0000x reference)
import jax
import jax.numpy as jnp
from jax.experimental import pallas as pl


def kernel(x, Wg, W1, W2):
    raise NotImplementedError("write your pallas kernel here")


# speedup vs baseline: 1.2751x; 1.2751x over previous
import functools

import jax
import jax.numpy as jnp
from jax.experimental import pallas as pl
from jax.experimental.pallas import tpu as pltpu

T = 2048
D = 1024
F = 2048
E = 8
K = 2

BT = 256
A = T * K
A_PAD = A + E * BT
NBLK = A_PAD // BT


def _cumsum_rows(m):
    c = m
    d = 1
    n = m.shape[0]
    while d < n:
        z = jnp.zeros((d,) + m.shape[1:], m.dtype)
        c = c + jnp.concatenate([z, c[:-d]], axis=0)
        d *= 2
    return c


def _router_body(x_ref, wg_ref, slots_ref, gates_ref, be_ref, act_ref):
    x = x_ref[...]
    wg = wg_ref[...]
    logits = jnp.dot(x, wg, preferred_element_type=jnp.float32)

    e_iota = jax.lax.broadcasted_iota(jnp.int32, (T, E), 1)
    neg_inf = jnp.float32(-jnp.inf)

    v0 = jnp.max(logits, axis=1, keepdims=True)
    is0 = logits == v0
    idx0 = jnp.min(jnp.where(is0, e_iota, E), axis=1)
    masked = jnp.where(e_iota == idx0[:, None], neg_inf, logits)
    v1 = jnp.max(masked, axis=1, keepdims=True)
    is1 = masked == v1
    idx1 = jnp.min(jnp.where(is1, e_iota, E), axis=1)

    g1 = 1.0 / (1.0 + jnp.exp(v0 - v1))
    g0 = 1.0 - g1

    mask0 = (e_iota == idx0[:, None]).astype(jnp.int32)
    mask1 = (e_iota == idx1[:, None]).astype(jnp.int32)
    cum0 = _cumsum_rows(mask0)
    cum1 = _cumsum_rows(mask1)
    total0 = cum0[-1:, :]
    count = total0 + cum1[-1:, :]

    pc = ((count + (BT - 1)) // BT) * BT
    pe = jax.lax.broadcasted_iota(jnp.int32, (E, E), 0)
    pe2 = jax.lax.broadcasted_iota(jnp.int32, (E, E), 1)
    pcb = jnp.broadcast_to(pc, (E, E))
    base_col = jnp.sum(jnp.where(pe2 < pe, pcb, 0), axis=1)
    base = base_col[None, :]

    rank0 = cum0 - 1
    rank1 = total0 + cum1 - 1
    slot0 = jnp.sum(mask0 * (base + rank0), axis=1)
    slot1 = jnp.sum(mask1 * (base + rank1), axis=1)

    slots_ref[0, :] = slot0
    slots_ref[1, :] = slot1
    gates_ref[0, :] = g0[:, 0]
    gates_ref[1, :] = g1[:, 0]

    blk = jax.lax.broadcasted_iota(jnp.int32, (NBLK, E), 0) * BT
    seg_end = jnp.broadcast_to(base + pc, (NBLK, E))
    be = jnp.sum((seg_end <= blk).astype(jnp.int32), axis=1)
    be_ref[...] = jnp.minimum(be, E - 1)
    total_padded = jnp.sum(pc)
    blk1 = jax.lax.iota(jnp.int32, NBLK) * BT
    act_ref[...] = (blk1 < total_padded).astype(jnp.int32)


def _router(x, Wg):
    return pl.pallas_call(
        _router_body,
        out_shape=(
            jax.ShapeDtypeStruct((K, T), jnp.int32),
            jax.ShapeDtypeStruct((K, T), jnp.float32),
            jax.ShapeDtypeStruct((NBLK,), jnp.int32),
            jax.ShapeDtypeStruct((NBLK,), jnp.int32),
        ),
    )(x, Wg)


def _expert_body(be_ref, act_ref, xg_ref, w1_ref, w2_ref, y_ref):
    i = pl.program_id(0)

    @pl.when(act_ref[i] == 1)
    def _():
        xb = xg_ref[...]
        h = jnp.dot(xb, w1_ref[0], preferred_element_type=jnp.float32)
        h = jax.nn.gelu(h)
        y_ref[...] = jnp.dot(h, w2_ref[0], preferred_element_type=jnp.float32)


def _experts(xg, W1, W2, be, act):
    grid_spec = pltpu.PrefetchScalarGridSpec(
        num_scalar_prefetch=2,
        grid=(NBLK,),
        in_specs=[
            pl.BlockSpec((BT, D), lambda i, be, act: (i, 0)),
            pl.BlockSpec((1, D, F), lambda i, be, act: (be[i], 0, 0)),
            pl.BlockSpec((1, F, D), lambda i, be, act: (be[i], 0, 0)),
        ],
        out_specs=pl.BlockSpec((BT, D), lambda i, be, act: (i, 0)),
    )
    return pl.pallas_call(
        _expert_body,
        grid_spec=grid_spec,
        out_shape=jax.ShapeDtypeStruct((A_PAD, D), jnp.float32),
    )(be, act, xg, W1, W2)


@jax.jit
def kernel(x, Wg, W1, W2):
    slots, gates, be, act = _router(x, Wg)
    slot0, slot1 = slots[0], slots[1]
    g0, g1 = gates[0], gates[1]

    xg = jnp.zeros((A_PAD, D), jnp.float32)
    xg = xg.at[slot0].set(x).at[slot1].set(x)

    y = _experts(xg, W1, W2, be, act)

    out = g0[:, None] * jnp.take(y, slot0, axis=0) \
        + g1[:, None] * jnp.take(y, slot1, axis=0)
    return out


# speedup vs baseline: 1.6616x; 1.3032x over previous
import functools

import jax
import jax.numpy as jnp
from jax import lax
from jax.experimental import pallas as pl
from jax.experimental.pallas import tpu as pltpu
from jax.experimental.pallas import tpu_sc as plsc

T = 2048
D = 1024
F = 2048
E = 8
K = 2

BT = 256
A = T * K
A_PAD = A + E * BT
NBLK = A_PAD // BT

NC = 2
NS = 16
NW = NC * NS
TOK_W = T // NW


def _cumsum_lanes(m):
    c = m
    d = 1
    n = m.shape[1]
    while d < n:
        z = jnp.zeros((m.shape[0], d), m.dtype)
        c = c + jnp.concatenate([z, c[:, :-d]], axis=1)
        d *= 2
    return c


def _router_body(x_ref, wg_ref, slots_ref, gates_ref, be_ref, act_ref):
    x = x_ref[...]
    wg = wg_ref[...]
    logits = jnp.dot(x, wg, preferred_element_type=jnp.float32)
    lt = logits.T

    er = jax.lax.broadcasted_iota(jnp.int32, (E, T), 0)
    neg_inf = jnp.float32(-jnp.inf)

    v0 = jnp.max(lt, axis=0, keepdims=True)
    idx0 = jnp.min(jnp.where(lt == v0, er, E), axis=0, keepdims=True)
    m0 = er == idx0
    masked = jnp.where(m0, neg_inf, lt)
    v1 = jnp.max(masked, axis=0, keepdims=True)
    idx1 = jnp.min(jnp.where(masked == v1, er, E), axis=0, keepdims=True)
    m1 = er == idx1

    g1 = 1.0 / (1.0 + jnp.exp(v0 - v1))
    g0 = 1.0 - g1

    mi0 = m0.astype(jnp.int32)
    mi1 = m1.astype(jnp.int32)
    cum0 = _cumsum_lanes(mi0)
    cum1 = _cumsum_lanes(mi1)
    tot0 = cum0[:, -1:]
    count = tot0 + cum1[:, -1:]

    pc = ((count + (BT - 1)) // BT) * BT
    b = pc
    d = 1
    while d < E:
        b = b + jnp.concatenate([jnp.zeros((d, 1), jnp.int32), b[:-d, :]],
                                axis=0)
        d *= 2
    base = b - pc

    rank0 = cum0 - 1
    rank1 = tot0 + cum1 - 1
    slot0 = jnp.sum(mi0 * (base + rank0), axis=0, keepdims=True)
    slot1 = jnp.sum(mi1 * (base + rank1), axis=0, keepdims=True)

    slots_ref[0:1, :] = slot0
    slots_ref[1:2, :] = slot1
    gates_ref[0:1, :] = g0
    gates_ref[1:2, :] = g1

    blk = jax.lax.broadcasted_iota(jnp.int32, (NBLK, E), 0) * BT
    seg_end = jnp.broadcast_to((base + pc).reshape(1, E), (NBLK, E))
    be = jnp.sum((seg_end <= blk).astype(jnp.int32), axis=1)
    be_ref[...] = jnp.minimum(be, E - 1)
    total_padded = jnp.sum(pc)
    blk1 = jax.lax.iota(jnp.int32, NBLK) * BT
    act_ref[...] = (blk1 < total_padded).astype(jnp.int32)


def _router(x, Wg):
    return pl.pallas_call(
        _router_body,
        out_shape=(
            jax.ShapeDtypeStruct((K, T), jnp.int32),
            jax.ShapeDtypeStruct((K, T), jnp.float32),
            jax.ShapeDtypeStruct((NBLK,), jnp.int32),
            jax.ShapeDtypeStruct((NBLK,), jnp.int32),
        ),
    )(x, Wg)


@functools.partial(
    pl.kernel,
    out_type=jax.ShapeDtypeStruct((A_PAD, D), jnp.float32),
    mesh=plsc.VectorSubcoreMesh(core_axis_name="c", subcore_axis_name="s",
                                num_cores=NC, num_subcores=NS),
    scratch_types=[
        pltpu.VMEM((TOK_W,), jnp.int32),
        pltpu.VMEM((TOK_W,), jnp.int32),
        pltpu.VMEM((TOK_W, D), jnp.float32),
        pltpu.SemaphoreType.DMA,
    ],
)
def _dispatch(x_hbm, slot0_hbm, slot1_hbm, xg_hbm, idx0_v, idx1_v, rows_v,
              sem):
    wid = lax.axis_index("s") * NC + lax.axis_index("c")
    base = wid * TOK_W
    pltpu.sync_copy(x_hbm.at[pl.ds(base, TOK_W)], rows_v)
    pltpu.sync_copy(slot0_hbm.at[pl.ds(base, TOK_W)], idx0_v)
    pltpu.sync_copy(slot1_hbm.at[pl.ds(base, TOK_W)], idx1_v)
    c0 = pltpu.async_copy(rows_v, xg_hbm.at[idx0_v], sem)
    c1 = pltpu.async_copy(rows_v, xg_hbm.at[idx1_v], sem)
    c0.wait()
    c1.wait()


def _expert_body(be_ref, act_ref, xg_ref, w1_ref, w2_ref, y_ref):
    i = pl.program_id(0)

    @pl.when(act_ref[i] == 1)
    def _():
        xb = xg_ref[...]
        h = jnp.dot(xb, w1_ref[0], preferred_element_type=jnp.float32)
        h = jax.nn.gelu(h)
        y_ref[...] = jnp.dot(h, w2_ref[0], preferred_element_type=jnp.float32)


def _experts(xg, W1, W2, be, act):
    grid_spec = pltpu.PrefetchScalarGridSpec(
        num_scalar_prefetch=2,
        grid=(NBLK,),
        in_specs=[
            pl.BlockSpec((BT, D), lambda i, be, act: (i, 0)),
            pl.BlockSpec((1, D, F), lambda i, be, act: (be[i], 0, 0)),
            pl.BlockSpec((1, F, D), lambda i, be, act: (be[i], 0, 0)),
        ],
        out_specs=pl.BlockSpec((BT, D), lambda i, be, act: (i, 0)),
    )
    return pl.pallas_call(
        _expert_body,
        grid_spec=grid_spec,
        out_shape=jax.ShapeDtypeStruct((A_PAD, D), jnp.float32),
    )(be, act, xg, W1, W2)


@jax.jit
def kernel(x, Wg, W1, W2):
    slots, gates, be, act = _router(x, Wg)
    slot0, slot1 = slots[0], slots[1]
    g0, g1 = gates[0], gates[1]

    xg = _dispatch(x, slot0, slot1)
    y = _experts(xg, W1, W2, be, act)

    out = g0[:, None] * jnp.take(y, slot0, axis=0) \
        + g1[:, None] * jnp.take(y, slot1, axis=0)
    return out
